# baseline (speedup 1.0000x reference)
import functools

import jax
import jax.numpy as jnp
from jax import lax
from jax.experimental import pallas as pl
from jax.experimental.pallas import tpu as pltpu
from jax.experimental.pallas import tpu_sc as plsc

N = 10000
NPAD = 10240
IN_SIZE = 256
HID = 512
OUT_SIZE = 256
C = 4
FC = 128
E = 160000
TILES = 16
EB = 128
NB = 80
NBH = NB // 2
EPT = NB * EB
EPAD = TILES * EPT
RPT = NPAD // TILES

_MESH = plsc.VectorSubcoreMesh(core_axis_name="c", subcore_axis_name="s")


def _seg_body(h3d, srcr, dstr, zeros_hbm, o3d,
              src_s, dst_s, bufs, acc, semg, sems):
    cid = lax.axis_index("c")
    sid = lax.axis_index("s")
    row0 = sid * RPT

    def process(hc, outc):
        pltpu.sync_copy(zeros_hbm.at[pl.ds(row0, RPT)], acc.at[pl.ds(row0, RPT)])
        plsc.subcore_barrier()

        def phase(j, p):
            pn = 1 - p

            @pl.when(j > 0)
            def _():
                pltpu.make_async_copy(
                    bufs.at[pn], acc.at[dst_s.at[j - 1]], sems).wait()

            @pl.when(j + 1 < NBH)
            def _():
                pltpu.async_copy(hc.at[src_s.at[j + 1]], bufs.at[pn],
                                 semg.at[pn])

            pltpu.make_async_copy(hc.at[src_s.at[j]], bufs.at[p], semg.at[p]).wait()
            pltpu.async_copy(bufs.at[p], acc.at[dst_s.at[j]], sems, add=True)

        def dbl(i, carry):
            phase(i * 2, 0)
            phase(i * 2 + 1, 1)
            return carry

        for hh in range(2):
            pltpu.sync_copy(srcr.at[sid, hh], src_s)
            pltpu.sync_copy(dstr.at[sid, hh], dst_s)
            pltpu.async_copy(hc.at[src_s.at[0]], bufs.at[0], semg.at[0])
            lax.fori_loop(0, NBH // 2, dbl, 0, unroll=False)
            pltpu.make_async_copy(bufs.at[(NBH - 1) % 2],
                                  acc.at[dst_s.at[NBH - 1]], sems).wait()
        plsc.subcore_barrier()
        pltpu.sync_copy(acc.at[pl.ds(row0, RPT)], outc.at[pl.ds(row0, RPT)])
        plsc.subcore_barrier()

    for k in range(2):
        chunk = cid * 2 + k
        process(h3d.at[chunk], o3d.at[chunk])


@functools.partial(jax.jit, donate_argnums=())
def _segment_sum_sc(hc, srcr, dstr, zeros_hbm):
    return pl.kernel(
        _seg_body,
        out_type=jax.ShapeDtypeStruct((C, NPAD, FC), jnp.float32),
        mesh=_MESH,
        scratch_types=[
            pltpu.VMEM((NBH, EB), jnp.int32),
            pltpu.VMEM((NBH, EB), jnp.int32),
            pltpu.VMEM((2, EB, FC), jnp.float32),
            pltpu.VMEM_SHARED((NPAD, FC), jnp.float32),
            pltpu.SemaphoreType.DMA((2,)),
            pltpu.SemaphoreType.DMA,
        ],
    )(hc, srcr, dstr, zeros_hbm)


def _pre_body(x_ref, w_ref, b_ref, o_ref):
    acc = jnp.dot(x_ref[...], w_ref[...], preferred_element_type=jnp.float32)
    for c in range(C):
        o_ref[c] = jnp.maximum(acc[:, c * FC:(c + 1) * FC] + b_ref[c], 0.0)


def _mm_body(h_ref, w_ref, o_ref):
    acc = jnp.dot(h_ref[0], w_ref[0:FC, :], preferred_element_type=jnp.float32)
    for c in range(1, C):
        acc += jnp.dot(h_ref[c], w_ref[c * FC:(c + 1) * FC, :],
                       preferred_element_type=jnp.float32)
    for c in range(C):
        o_ref[c] = acc[:, c * FC:(c + 1) * FC]


def _madd_body(p_ref, g_ref, w_ref, b_ref, a_ref, o_ref):
    acc = jnp.dot(g_ref[0], w_ref[0:FC, :], preferred_element_type=jnp.float32)
    for c in range(1, C):
        acc += jnp.dot(g_ref[c], w_ref[c * FC:(c + 1) * FC, :],
                       preferred_element_type=jnp.float32)
    a = a_ref[0, 0]
    for c in range(C):
        v = p_ref[c] + acc[:, c * FC:(c + 1) * FC] + b_ref[c]
        o_ref[c] = jnp.where(v >= 0, v, a * v)


def _post_body(h_ref, w_ref, b_ref, o_ref):
    acc = jnp.dot(h_ref[0], w_ref[0:FC, :], preferred_element_type=jnp.float32)
    for c in range(1, C):
        acc += jnp.dot(h_ref[c], w_ref[c * FC:(c + 1) * FC, :],
                       preferred_element_type=jnp.float32)
    o_ref[...] = acc + b_ref[...]


_BR = 512
_GRID = (NPAD // _BR,)


def _pre_gemm(x, w, b):
    return pl.pallas_call(
        _pre_body,
        grid=_GRID,
        in_specs=[
            pl.BlockSpec((_BR, IN_SIZE), lambda i: (i, 0)),
            pl.BlockSpec((IN_SIZE, HID), lambda i: (0, 0)),
            pl.BlockSpec((C, FC), lambda i: (0, 0)),
        ],
        out_specs=pl.BlockSpec((C, _BR, FC), lambda i: (0, i, 0)),
        out_shape=jax.ShapeDtypeStruct((C, NPAD, FC), jnp.float32),
    )(x, w, b)


def _mm_gemm(h, w):
    return pl.pallas_call(
        _mm_body,
        grid=_GRID,
        in_specs=[
            pl.BlockSpec((C, _BR, FC), lambda i: (0, i, 0)),
            pl.BlockSpec((HID, HID), lambda i: (0, 0)),
        ],
        out_specs=pl.BlockSpec((C, _BR, FC), lambda i: (0, i, 0)),
        out_shape=jax.ShapeDtypeStruct((C, NPAD, FC), jnp.float32),
    )(h, w)


def _madd_gemm(p, g, w, b, a):
    return pl.pallas_call(
        _madd_body,
        grid=_GRID,
        in_specs=[
            pl.BlockSpec((C, _BR, FC), lambda i: (0, i, 0)),
            pl.BlockSpec((C, _BR, FC), lambda i: (0, i, 0)),
            pl.BlockSpec((HID, HID), lambda i: (0, 0)),
            pl.BlockSpec((C, FC), lambda i: (0, 0)),
            pl.BlockSpec(memory_space=pltpu.SMEM),
        ],
        out_specs=pl.BlockSpec((C, _BR, FC), lambda i: (0, i, 0)),
        out_shape=jax.ShapeDtypeStruct((C, NPAD, FC), jnp.float32),
    )(p, g, w, b, a)


def _post_gemm(h, w, b):
    return pl.pallas_call(
        _post_body,
        grid=_GRID,
        in_specs=[
            pl.BlockSpec((C, _BR, FC), lambda i: (0, i, 0)),
            pl.BlockSpec((HID, OUT_SIZE), lambda i: (0, 0)),
            pl.BlockSpec((1, OUT_SIZE), lambda i: (0, 0)),
        ],
        out_specs=pl.BlockSpec((_BR, OUT_SIZE), lambda i: (i, 0)),
        out_shape=jax.ShapeDtypeStruct((NPAD, OUT_SIZE), jnp.float32),
    )(h, w, b)


def kernel(features, edge_index, W_pre, b_pre, W1, b1, a1, W2, b2, a2, W_post, b_post):
    x = jnp.pad(features, ((0, NPAD - N), (0, 0)))
    src = edge_index[0].astype(jnp.int32)
    dst = edge_index[1].astype(jnp.int32)
    npad_e = EPAD - E
    pad_src = (jnp.arange(npad_e, dtype=jnp.int32) * 97) % N
    pad_dst = N + (jnp.arange(npad_e, dtype=jnp.int32) % (NPAD - N))
    srcr = jnp.concatenate([src, pad_src]).reshape(TILES, 2, NBH, EB)
    dstr = jnp.concatenate([dst, pad_dst]).reshape(TILES, 2, NBH, EB)
    zeros_hbm = jnp.zeros((NPAD, FC), jnp.float32)

    b_pre_c = b_pre.reshape(C, FC)
    b1_c = b1.reshape(C, FC)
    b2_c = b2.reshape(C, FC)
    a1_s = a1.reshape(1, 1)
    a2_s = a2.reshape(1, 1)

    h = _pre_gemm(x, W_pre, b_pre_c)
    g = _segment_sum_sc(h, srcr, dstr, zeros_hbm)
    p = _mm_gemm(h, W1)
    h = _madd_gemm(p, g, W1, b1_c, a1_s)
    g = _segment_sum_sc(h, srcr, dstr, zeros_hbm)
    p = _mm_gemm(h, W2)
    h = _madd_gemm(p, g, W2, b2_c, a2_s)
    out = _post_gemm(h, W_post, b_post.reshape(1, OUT_SIZE))
    return out[:N]


# speedup vs baseline: 1.0202x; 1.0202x over previous
import functools

import jax
import jax.numpy as jnp
from jax import lax
from jax.experimental import pallas as pl
from jax.experimental.pallas import tpu as pltpu
from jax.experimental.pallas import tpu_sc as plsc

N = 10000
NPAD = 10240
IN_SIZE = 256
HID = 512
OUT_SIZE = 256
C = 4
FC = 128
E = 160000
TILES = 16
EB = 128
NB = 80
NBH = NB // 2
EPT = NB * EB
EPAD = TILES * EPT
RPT = NPAD // TILES

_MESH = plsc.VectorSubcoreMesh(core_axis_name="c", subcore_axis_name="s")


def _seg_body(h3d, srcr, dstr, zeros_hbm, o3d,
              src_s, dst_s, bufs, acc, semg, sems):
    cid = lax.axis_index("c")
    sid = lax.axis_index("s")
    row0 = sid * RPT

    def process(hc, outc):
        pltpu.sync_copy(zeros_hbm.at[pl.ds(row0, RPT)], acc.at[pl.ds(row0, RPT)])
        plsc.subcore_barrier()

        def phase(j, p):
            pn = 1 - p

            @pl.when(j > 0)
            def _():
                pltpu.make_async_copy(
                    bufs.at[pn], acc.at[dst_s.at[j - 1]], sems).wait()

            @pl.when(j + 1 < NBH)
            def _():
                pltpu.async_copy(hc.at[src_s.at[j + 1]], bufs.at[pn],
                                 semg.at[pn])

            pltpu.make_async_copy(hc.at[src_s.at[j]], bufs.at[p], semg.at[p]).wait()
            pltpu.async_copy(bufs.at[p], acc.at[dst_s.at[j]], sems, add=True)

        def dbl(i, carry):
            phase(i * 2, 0)
            phase(i * 2 + 1, 1)
            return carry

        for hh in range(2):
            pltpu.sync_copy(srcr.at[sid, hh], src_s)
            pltpu.sync_copy(dstr.at[sid, hh], dst_s)
            pltpu.async_copy(hc.at[src_s.at[0]], bufs.at[0], semg.at[0])
            lax.fori_loop(0, NBH // 2, dbl, 0, unroll=False)
            pltpu.make_async_copy(bufs.at[(NBH - 1) % 2],
                                  acc.at[dst_s.at[NBH - 1]], sems).wait()
        plsc.subcore_barrier()
        pltpu.sync_copy(acc.at[pl.ds(row0, RPT)], outc.at[pl.ds(row0, RPT)])
        plsc.subcore_barrier()

    for k in range(2):
        chunk = cid * 2 + k
        process(h3d.at[chunk], o3d.at[chunk])


@functools.partial(jax.jit, donate_argnums=())
def _segment_sum_sc(hc, srcr, dstr, zeros_hbm):
    return pl.kernel(
        _seg_body,
        out_type=jax.ShapeDtypeStruct((C, NPAD, FC), jnp.float32),
        mesh=_MESH,
        scratch_types=[
            pltpu.VMEM((NBH, EB), jnp.int32),
            pltpu.VMEM((NBH, EB), jnp.int32),
            pltpu.VMEM((2, EB, FC), jnp.float32),
            pltpu.VMEM_SHARED((NPAD, FC), jnp.float32),
            pltpu.SemaphoreType.DMA((2,)),
            pltpu.SemaphoreType.DMA,
        ],
    )(hc, srcr, dstr, zeros_hbm)


def _pre_body(x_ref, w_ref, b_ref, o_ref):
    acc = jnp.dot(x_ref[...], w_ref[...], preferred_element_type=jnp.float32)
    for c in range(C):
        o_ref[c] = jnp.maximum(acc[:, c * FC:(c + 1) * FC] + b_ref[c], 0.0)


def _mid_body(h_ref, g_ref, w_ref, b_ref, a_ref, o_ref):
    s = h_ref[...] + g_ref[...]
    acc = jnp.dot(s[0], w_ref[0:FC, :], preferred_element_type=jnp.float32)
    for c in range(1, C):
        acc += jnp.dot(s[c], w_ref[c * FC:(c + 1) * FC, :],
                       preferred_element_type=jnp.float32)
    a = a_ref[0, 0]
    for c in range(C):
        v = acc[:, c * FC:(c + 1) * FC] + b_ref[c]
        o_ref[c] = jnp.where(v >= 0, v, a * v)


def _post_body(h_ref, w_ref, b_ref, o_ref):
    acc = jnp.dot(h_ref[0], w_ref[0:FC, :], preferred_element_type=jnp.float32)
    for c in range(1, C):
        acc += jnp.dot(h_ref[c], w_ref[c * FC:(c + 1) * FC, :],
                       preferred_element_type=jnp.float32)
    o_ref[...] = acc + b_ref[...]


_BR = 512
_GRID = (NPAD // _BR,)


def _pre_gemm(x, w, b):
    return pl.pallas_call(
        _pre_body,
        grid=_GRID,
        in_specs=[
            pl.BlockSpec((_BR, IN_SIZE), lambda i: (i, 0)),
            pl.BlockSpec((IN_SIZE, HID), lambda i: (0, 0)),
            pl.BlockSpec((C, FC), lambda i: (0, 0)),
        ],
        out_specs=pl.BlockSpec((C, _BR, FC), lambda i: (0, i, 0)),
        out_shape=jax.ShapeDtypeStruct((C, NPAD, FC), jnp.float32),
    )(x, w, b)


def _mid_gemm(h, g, w, b, a):
    return pl.pallas_call(
        _mid_body,
        grid=_GRID,
        in_specs=[
            pl.BlockSpec((C, _BR, FC), lambda i: (0, i, 0)),
            pl.BlockSpec((C, _BR, FC), lambda i: (0, i, 0)),
            pl.BlockSpec((HID, HID), lambda i: (0, 0)),
            pl.BlockSpec((C, FC), lambda i: (0, 0)),
            pl.BlockSpec(memory_space=pltpu.SMEM),
        ],
        out_specs=pl.BlockSpec((C, _BR, FC), lambda i: (0, i, 0)),
        out_shape=jax.ShapeDtypeStruct((C, NPAD, FC), jnp.float32),
    )(h, g, w, b, a)


def _post_gemm(h, w, b):
    return pl.pallas_call(
        _post_body,
        grid=_GRID,
        in_specs=[
            pl.BlockSpec((C, _BR, FC), lambda i: (0, i, 0)),
            pl.BlockSpec((HID, OUT_SIZE), lambda i: (0, 0)),
            pl.BlockSpec((1, OUT_SIZE), lambda i: (0, 0)),
        ],
        out_specs=pl.BlockSpec((_BR, OUT_SIZE), lambda i: (i, 0)),
        out_shape=jax.ShapeDtypeStruct((NPAD, OUT_SIZE), jnp.float32),
    )(h, w, b)


def kernel(features, edge_index, W_pre, b_pre, W1, b1, a1, W2, b2, a2, W_post, b_post):
    x = jnp.pad(features, ((0, NPAD - N), (0, 0)))
    src = edge_index[0].astype(jnp.int32)
    dst = edge_index[1].astype(jnp.int32)
    npad_e = EPAD - E
    pad_src = (jnp.arange(npad_e, dtype=jnp.int32) * 97) % N
    pad_dst = N + (jnp.arange(npad_e, dtype=jnp.int32) % (NPAD - N))
    srcr = jnp.concatenate([src, pad_src]).reshape(TILES, 2, NBH, EB)
    dstr = jnp.concatenate([dst, pad_dst]).reshape(TILES, 2, NBH, EB)
    zeros_hbm = jnp.zeros((NPAD, FC), jnp.float32)

    b_pre_c = b_pre.reshape(C, FC)
    b1_c = b1.reshape(C, FC)
    b2_c = b2.reshape(C, FC)
    a1_s = a1.reshape(1, 1)
    a2_s = a2.reshape(1, 1)

    h = _pre_gemm(x, W_pre, b_pre_c)
    g = _segment_sum_sc(h, srcr, dstr, zeros_hbm)
    h = _mid_gemm(h, g, W1, b1_c, a1_s)
    g = _segment_sum_sc(h, srcr, dstr, zeros_hbm)
    h = _mid_gemm(h, g, W2, b2_c, a2_s)
    out = _post_gemm(h, W_post, b_post.reshape(1, OUT_SIZE))
    return out[:N]


# speedup vs baseline: 1.0539x; 1.0331x over previous
import functools

import jax
import jax.numpy as jnp
from jax import lax
from jax.experimental import pallas as pl
from jax.experimental.pallas import tpu as pltpu
from jax.experimental.pallas import tpu_sc as plsc

N = 10000
NPAD = 10240
IN_SIZE = 256
HID = 512
OUT_SIZE = 256
C = 4
FC = 128
E = 160000
TILES = 16
EB = 128
NB = 80
NBH = NB // 2
EPT = NB * EB
EPAD = TILES * EPT
RPT = NPAD // TILES

_MESH = plsc.VectorSubcoreMesh(core_axis_name="c", subcore_axis_name="s")


def _seg_body(h3d, srcr, dstr, zeros_hbm, o3d,
              src_s, dst_s, bufs, acc, semg, sems):
    cid = lax.axis_index("c")
    sid = lax.axis_index("s")
    row0 = sid * RPT

    def process(hc, outc):
        pltpu.sync_copy(zeros_hbm.at[pl.ds(row0, RPT)], acc.at[pl.ds(row0, RPT)])
        plsc.subcore_barrier()

        def phase(j, p):
            pn = 1 - p

            @pl.when(j > 0)
            def _():
                pltpu.make_async_copy(
                    bufs.at[pn], acc.at[dst_s.at[j - 1]], sems).wait()

            @pl.when(j + 1 < NBH)
            def _():
                pltpu.async_copy(hc.at[src_s.at[j + 1]], bufs.at[pn],
                                 semg.at[pn])

            pltpu.make_async_copy(hc.at[src_s.at[j]], bufs.at[p], semg.at[p]).wait()
            pltpu.async_copy(bufs.at[p], acc.at[dst_s.at[j]], sems, add=True)

        def dbl(i, carry):
            phase(i * 2, 0)
            phase(i * 2 + 1, 1)
            return carry

        for hh in range(2):
            pltpu.sync_copy(srcr.at[sid, hh], src_s)
            pltpu.sync_copy(dstr.at[sid, hh], dst_s)
            pltpu.async_copy(hc.at[src_s.at[0]], bufs.at[0], semg.at[0])
            lax.fori_loop(0, NBH // 2, dbl, 0, unroll=False)
            pltpu.make_async_copy(bufs.at[(NBH - 1) % 2],
                                  acc.at[dst_s.at[NBH - 1]], sems).wait()
        plsc.subcore_barrier()
        pltpu.sync_copy(acc.at[pl.ds(row0, RPT)], outc.at[pl.ds(row0, RPT)])
        plsc.subcore_barrier()

    for k in range(2):
        chunk = cid * 2 + k
        process(h3d.at[chunk], o3d.at[chunk])


@functools.partial(jax.jit, donate_argnums=())
def _segment_sum_sc(hc, srcr, dstr, zeros_hbm):
    return pl.kernel(
        _seg_body,
        out_type=jax.ShapeDtypeStruct((C, NPAD, FC), jnp.float32),
        mesh=_MESH,
        scratch_types=[
            pltpu.VMEM((NBH, EB), jnp.int32),
            pltpu.VMEM((NBH, EB), jnp.int32),
            pltpu.VMEM((2, EB, FC), jnp.float32),
            pltpu.VMEM_SHARED((NPAD, FC), jnp.float32),
            pltpu.SemaphoreType.DMA((2,)),
            pltpu.SemaphoreType.DMA,
        ],
    )(hc, srcr, dstr, zeros_hbm)


def _pre_body(x_ref, w_ref, b_ref, o_ref):
    acc = jnp.dot(x_ref[...], w_ref[...], preferred_element_type=jnp.float32)
    for c in range(C):
        o_ref[c] = jnp.maximum(acc[:, c * FC:(c + 1) * FC] + b_ref[c], 0.0)


def _mid_body(h_ref, g_ref, w_ref, b_ref, a_ref, o_ref):
    s = h_ref[...] + g_ref[...]
    acc = jnp.dot(s[0], w_ref[0:FC, :], preferred_element_type=jnp.float32)
    for c in range(1, C):
        acc += jnp.dot(s[c], w_ref[c * FC:(c + 1) * FC, :],
                       preferred_element_type=jnp.float32)
    a = a_ref[0, 0]
    for c in range(C):
        v = acc[:, c * FC:(c + 1) * FC] + b_ref[c]
        o_ref[c] = jnp.where(v >= 0, v, a * v)


def _post_body(h_ref, w_ref, b_ref, o_ref):
    acc = jnp.dot(h_ref[0], w_ref[0:FC, :], preferred_element_type=jnp.float32)
    for c in range(1, C):
        acc += jnp.dot(h_ref[c], w_ref[c * FC:(c + 1) * FC, :],
                       preferred_element_type=jnp.float32)
    o_ref[...] = acc + b_ref[...]


_BR = 1024
_GRID = (NPAD // _BR,)


def _pre_gemm(x, w, b):
    return pl.pallas_call(
        _pre_body,
        grid=_GRID,
        in_specs=[
            pl.BlockSpec((_BR, IN_SIZE), lambda i: (i, 0)),
            pl.BlockSpec((IN_SIZE, HID), lambda i: (0, 0)),
            pl.BlockSpec((C, FC), lambda i: (0, 0)),
        ],
        out_specs=pl.BlockSpec((C, _BR, FC), lambda i: (0, i, 0)),
        out_shape=jax.ShapeDtypeStruct((C, NPAD, FC), jnp.float32),
    )(x, w, b)


def _mid_gemm(h, g, w, b, a):
    return pl.pallas_call(
        _mid_body,
        grid=_GRID,
        in_specs=[
            pl.BlockSpec((C, _BR, FC), lambda i: (0, i, 0)),
            pl.BlockSpec((C, _BR, FC), lambda i: (0, i, 0)),
            pl.BlockSpec((HID, HID), lambda i: (0, 0)),
            pl.BlockSpec((C, FC), lambda i: (0, 0)),
            pl.BlockSpec(memory_space=pltpu.SMEM),
        ],
        out_specs=pl.BlockSpec((C, _BR, FC), lambda i: (0, i, 0)),
        out_shape=jax.ShapeDtypeStruct((C, NPAD, FC), jnp.float32),
    )(h, g, w, b, a)


def _post_gemm(h, w, b):
    return pl.pallas_call(
        _post_body,
        grid=_GRID,
        in_specs=[
            pl.BlockSpec((C, _BR, FC), lambda i: (0, i, 0)),
            pl.BlockSpec((HID, OUT_SIZE), lambda i: (0, 0)),
            pl.BlockSpec((1, OUT_SIZE), lambda i: (0, 0)),
        ],
        out_specs=pl.BlockSpec((_BR, OUT_SIZE), lambda i: (i, 0)),
        out_shape=jax.ShapeDtypeStruct((NPAD, OUT_SIZE), jnp.float32),
    )(h, w, b)


def kernel(features, edge_index, W_pre, b_pre, W1, b1, a1, W2, b2, a2, W_post, b_post):
    x = jnp.pad(features, ((0, NPAD - N), (0, 0)))
    src = edge_index[0].astype(jnp.int32)
    dst = edge_index[1].astype(jnp.int32)
    npad_e = EPAD - E
    pad_src = (jnp.arange(npad_e, dtype=jnp.int32) * 97) % N
    pad_dst = N + (jnp.arange(npad_e, dtype=jnp.int32) % (NPAD - N))
    srcr = jnp.concatenate([src, pad_src]).reshape(TILES, 2, NBH, EB)
    dstr = jnp.concatenate([dst, pad_dst]).reshape(TILES, 2, NBH, EB)
    zeros_hbm = jnp.zeros((NPAD, FC), jnp.float32)

    b_pre_c = b_pre.reshape(C, FC)
    b1_c = b1.reshape(C, FC)
    b2_c = b2.reshape(C, FC)
    a1_s = a1.reshape(1, 1)
    a2_s = a2.reshape(1, 1)

    h = _pre_gemm(x, W_pre, b_pre_c)
    g = _segment_sum_sc(h, srcr, dstr, zeros_hbm)
    h = _mid_gemm(h, g, W1, b1_c, a1_s)
    g = _segment_sum_sc(h, srcr, dstr, zeros_hbm)
    h = _mid_gemm(h, g, W2, b2_c, a2_s)
    out = _post_gemm(h, W_post, b_post.reshape(1, OUT_SIZE))
    return out[:N]


# speedup vs baseline: 1.0940x; 1.0380x over previous
import functools

import jax
import jax.numpy as jnp
from jax import lax
from jax.experimental import pallas as pl
from jax.experimental.pallas import tpu as pltpu
from jax.experimental.pallas import tpu_sc as plsc

N = 10000
NPAD = 10240
IN_SIZE = 256
HID = 512
OUT_SIZE = 256
C = 4
FC = 128
E = 160000
TILES = 16
EB = 64
NB = 160
NBH = NB // 4
EPT = NB * EB
EPAD = TILES * EPT
RPT = NPAD // TILES

_MESH = plsc.VectorSubcoreMesh(core_axis_name="c", subcore_axis_name="s")


def _seg_body(h3d, srcr, dstr, zeros_hbm, o3d,
              src_s, dst_s, bufs, acc, semg, sems):
    cid = lax.axis_index("c")
    sid = lax.axis_index("s")
    row0 = sid * RPT

    def process(hc, outc):
        pltpu.sync_copy(zeros_hbm.at[pl.ds(row0, RPT)], acc.at[pl.ds(row0, RPT)])
        plsc.subcore_barrier()

        def phase(j, q):
            qf = (q + 3) % 4

            @pl.when(j > 0)
            def _():
                pltpu.make_async_copy(
                    bufs.at[qf], acc.at[dst_s.at[j - 1]], sems).wait()

            @pl.when(j + 3 < NBH)
            def _():
                pltpu.async_copy(hc.at[src_s.at[j + 3]], bufs.at[qf],
                                 semg.at[qf])

            pltpu.make_async_copy(hc.at[src_s.at[j]], bufs.at[q], semg.at[q]).wait()
            pltpu.async_copy(bufs.at[q], acc.at[dst_s.at[j]], sems, add=True)

        def quad(i, carry):
            for q in range(4):
                phase(i * 4 + q, q)
            return carry

        for hh in range(4):
            pltpu.sync_copy(srcr.at[sid, hh], src_s)
            pltpu.sync_copy(dstr.at[sid, hh], dst_s)
            for b in range(3):
                pltpu.async_copy(hc.at[src_s.at[b]], bufs.at[b], semg.at[b])
            lax.fori_loop(0, NBH // 4, quad, 0, unroll=False)
            pltpu.make_async_copy(bufs.at[(NBH - 1) % 4],
                                  acc.at[dst_s.at[NBH - 1]], sems).wait()
        plsc.subcore_barrier()
        pltpu.sync_copy(acc.at[pl.ds(row0, RPT)], outc.at[pl.ds(row0, RPT)])
        plsc.subcore_barrier()

    for k in range(2):
        chunk = cid * 2 + k
        process(h3d.at[chunk], o3d.at[chunk])


@functools.partial(jax.jit, donate_argnums=())
def _segment_sum_sc(hc, srcr, dstr, zeros_hbm):
    return pl.kernel(
        _seg_body,
        out_type=jax.ShapeDtypeStruct((C, NPAD, FC), jnp.float32),
        mesh=_MESH,
        scratch_types=[
            pltpu.VMEM((NBH, EB), jnp.int32),
            pltpu.VMEM((NBH, EB), jnp.int32),
            pltpu.VMEM((4, EB, FC), jnp.float32),
            pltpu.VMEM_SHARED((NPAD, FC), jnp.float32),
            pltpu.SemaphoreType.DMA((4,)),
            pltpu.SemaphoreType.DMA,
        ],
    )(hc, srcr, dstr, zeros_hbm)


def _pre_body(x_ref, w_ref, b_ref, o_ref):
    acc = jnp.dot(x_ref[...], w_ref[...], preferred_element_type=jnp.float32)
    for c in range(C):
        o_ref[c] = jnp.maximum(acc[:, c * FC:(c + 1) * FC] + b_ref[c], 0.0)


def _mid_body(h_ref, g_ref, w_ref, b_ref, a_ref, o_ref):
    s = h_ref[...] + g_ref[...]
    acc = jnp.dot(s[0], w_ref[0:FC, :], preferred_element_type=jnp.float32)
    for c in range(1, C):
        acc += jnp.dot(s[c], w_ref[c * FC:(c + 1) * FC, :],
                       preferred_element_type=jnp.float32)
    a = a_ref[0, 0]
    for c in range(C):
        v = acc[:, c * FC:(c + 1) * FC] + b_ref[c]
        o_ref[c] = jnp.where(v >= 0, v, a * v)


def _post_body(h_ref, w_ref, b_ref, o_ref):
    acc = jnp.dot(h_ref[0], w_ref[0:FC, :], preferred_element_type=jnp.float32)
    for c in range(1, C):
        acc += jnp.dot(h_ref[c], w_ref[c * FC:(c + 1) * FC, :],
                       preferred_element_type=jnp.float32)
    o_ref[...] = acc + b_ref[...]


_BR = 1024
_GRID = (NPAD // _BR,)


def _pre_gemm(x, w, b):
    return pl.pallas_call(
        _pre_body,
        grid=_GRID,
        in_specs=[
            pl.BlockSpec((_BR, IN_SIZE), lambda i: (i, 0)),
            pl.BlockSpec((IN_SIZE, HID), lambda i: (0, 0)),
            pl.BlockSpec((C, FC), lambda i: (0, 0)),
        ],
        out_specs=pl.BlockSpec((C, _BR, FC), lambda i: (0, i, 0)),
        out_shape=jax.ShapeDtypeStruct((C, NPAD, FC), jnp.float32),
    )(x, w, b)


def _mid_gemm(h, g, w, b, a):
    return pl.pallas_call(
        _mid_body,
        grid=_GRID,
        in_specs=[
            pl.BlockSpec((C, _BR, FC), lambda i: (0, i, 0)),
            pl.BlockSpec((C, _BR, FC), lambda i: (0, i, 0)),
            pl.BlockSpec((HID, HID), lambda i: (0, 0)),
            pl.BlockSpec((C, FC), lambda i: (0, 0)),
            pl.BlockSpec(memory_space=pltpu.SMEM),
        ],
        out_specs=pl.BlockSpec((C, _BR, FC), lambda i: (0, i, 0)),
        out_shape=jax.ShapeDtypeStruct((C, NPAD, FC), jnp.float32),
    )(h, g, w, b, a)


def _post_gemm(h, w, b):
    return pl.pallas_call(
        _post_body,
        grid=_GRID,
        in_specs=[
            pl.BlockSpec((C, _BR, FC), lambda i: (0, i, 0)),
            pl.BlockSpec((HID, OUT_SIZE), lambda i: (0, 0)),
            pl.BlockSpec((1, OUT_SIZE), lambda i: (0, 0)),
        ],
        out_specs=pl.BlockSpec((_BR, OUT_SIZE), lambda i: (i, 0)),
        out_shape=jax.ShapeDtypeStruct((NPAD, OUT_SIZE), jnp.float32),
    )(h, w, b)


def kernel(features, edge_index, W_pre, b_pre, W1, b1, a1, W2, b2, a2, W_post, b_post):
    x = jnp.pad(features, ((0, NPAD - N), (0, 0)))
    src = edge_index[0].astype(jnp.int32)
    dst = edge_index[1].astype(jnp.int32)
    npad_e = EPAD - E
    pad_src = (jnp.arange(npad_e, dtype=jnp.int32) * 97) % N
    pad_dst = N + (jnp.arange(npad_e, dtype=jnp.int32) % (NPAD - N))
    srcr = jnp.concatenate([src, pad_src]).reshape(TILES, 4, NBH, EB)
    dstr = jnp.concatenate([dst, pad_dst]).reshape(TILES, 4, NBH, EB)
    zeros_hbm = jnp.zeros((NPAD, FC), jnp.float32)

    b_pre_c = b_pre.reshape(C, FC)
    b1_c = b1.reshape(C, FC)
    b2_c = b2.reshape(C, FC)
    a1_s = a1.reshape(1, 1)
    a2_s = a2.reshape(1, 1)

    h = _pre_gemm(x, W_pre, b_pre_c)
    g = _segment_sum_sc(h, srcr, dstr, zeros_hbm)
    h = _mid_gemm(h, g, W1, b1_c, a1_s)
    g = _segment_sum_sc(h, srcr, dstr, zeros_hbm)
    h = _mid_gemm(h, g, W2, b2_c, a2_s)
    out = _post_gemm(h, W_post, b_post.reshape(1, OUT_SIZE))
    return out[:N]
